# baseline (speedup 1.0000x reference)
import functools

import jax
import jax.numpy as jnp
from jax import lax
from jax.experimental import pallas as pl
from jax.experimental.pallas import tpu as pltpu
from jax.experimental.pallas import tpu_sc as plsc

EPS = 1e-05
NC = 2
NS = 16
NW = NC * NS


def _sc_gather(uidx2, lidx2, pidx2, bidx2, ut2, it2, bias2):
    B = uidx2.shape[0] * uidx2.shape[1]
    bpw = B // NW
    nq = bpw // 128

    mesh = plsc.VectorSubcoreMesh(core_axis_name="c", subcore_axis_name="s",
                                  num_cores=NC, num_subcores=NS)

    @functools.partial(
        pl.kernel,
        out_type=(
            jax.ShapeDtypeStruct((B, 128), jnp.float32),
            jax.ShapeDtypeStruct((B, 128), jnp.float32),
            jax.ShapeDtypeStruct((B, 128), jnp.float32),
            jax.ShapeDtypeStruct((B, 128), jnp.float32),
        ),
        mesh=mesh,
        scratch_types=[
            pltpu.VMEM((nq, 128), jnp.int32),
            pltpu.VMEM((nq, 128), jnp.int32),
            pltpu.VMEM((nq, 128), jnp.int32),
            pltpu.VMEM((nq, 128), jnp.int32),
            pltpu.VMEM((128, 128), jnp.float32),
            pltpu.VMEM((128, 128), jnp.float32),
            pltpu.VMEM((128, 128), jnp.float32),
            pltpu.VMEM((128, 128), jnp.float32),
            pltpu.SemaphoreType.DMA,
        ],
    )
    def k(uidx_h, lidx_h, pidx_h, bidx_h, ut_h, it_h, bias_h,
          u_out, l_out, p_out, b_out,
          uq, lq, pq, bq, ubuf, lbuf, pbuf, bbuf, sem):
        wid = lax.axis_index("s") * NC + lax.axis_index("c")
        pltpu.sync_copy(uidx_h.at[pl.ds(wid * nq, nq)], uq)
        pltpu.sync_copy(lidx_h.at[pl.ds(wid * nq, nq)], lq)
        pltpu.sync_copy(pidx_h.at[pl.ds(wid * nq, nq)], pq)
        pltpu.sync_copy(bidx_h.at[pl.ds(wid * nq, nq)], bq)
        for q in range(nq):
            cps = [
                pltpu.async_copy(ut_h.at[uq.at[q]], ubuf, sem),
                pltpu.async_copy(it_h.at[lq.at[q]], lbuf, sem),
                pltpu.async_copy(it_h.at[pq.at[q]], pbuf, sem),
                pltpu.async_copy(bias_h.at[bq.at[q]], bbuf, sem),
            ]
            for cp in cps:
                cp.wait()
            row0 = wid * bpw + q * 128
            pltpu.sync_copy(ubuf, u_out.at[pl.ds(row0, 128)])
            pltpu.sync_copy(lbuf, l_out.at[pl.ds(row0, 128)])
            pltpu.sync_copy(pbuf, p_out.at[pl.ds(row0, 128)])
            pltpu.sync_copy(bbuf, b_out.at[pl.ds(row0, 128)])

    return k(uidx2, lidx2, pidx2, bidx2, ut2, it2, bias2)


def _tc_math_body(u_ref, l_ref, p_ref, b_ref, us_ref, ls_ref, ps_ref,
                  bl_ref, g_ref, o_ref):
    def ext(x, s):
        w0 = x[:, 0:32]
        w1 = x[:, 32:64]
        w2 = x[:, 64:96]
        w3 = x[:, 96:128]
        return jnp.where(s == 0, w0,
                         jnp.where(s == 1, w1, jnp.where(s == 2, w2, w3)))

    ue = ext(u_ref[...], us_ref[...])
    le = ext(l_ref[...], ls_ref[...])
    pe = ext(p_ref[...], ps_ref[...])
    lane = lax.broadcasted_iota(jnp.int32, (1, 128), 1)
    bias = jnp.sum(jnp.where(bl_ref[...] == lane, b_ref[...], 0.0),
                   axis=1, keepdims=True)
    v = ue + le + g_ref[...] + EPS
    p = pe + EPS
    s_v = jnp.sum(v * v, axis=1, keepdims=True)
    s_p = jnp.sum(p * p, axis=1, keepdims=True)
    vp = jnp.sum(v * p, axis=1, keepdims=True)
    nv = jnp.sqrt(s_v)
    np_ = jnp.sqrt(s_p)
    a = jnp.minimum(jnp.tanh(nv), 1.0 - EPS) / nv
    b = jnp.minimum(jnp.tanh(np_), 1.0 - EPS) / np_
    nx = jnp.clip(a * a * s_v, 0.0, 1.0 - 1e-06)
    ny = jnp.clip(b * b * s_p, 0.0, 1.0 - 1e-06)
    dd = jnp.maximum(a * a * s_v + b * b * s_p - 2.0 * a * b * vp, 0.0)
    t = 1.0 + 2.0 * (dd / ((1.0 - nx) * (1.0 - ny)))
    dist = jnp.log(t + jnp.sqrt(jnp.maximum(t * t - 1.0, 0.0)))
    o_ref[...] = -dist + bias


def _tc_math(u128, l128, p128, b128, usel, lsel, psel, blane, gt):
    B = u128.shape[0]
    blk = 512
    grid = (B // blk,)
    row_spec = pl.BlockSpec((blk, 128), lambda i: (i, 0))
    col_spec = pl.BlockSpec((blk, 1), lambda i: (i, 0))
    return pl.pallas_call(
        _tc_math_body,
        grid=grid,
        in_specs=[row_spec, row_spec, row_spec, row_spec,
                  col_spec, col_spec, col_spec, col_spec,
                  pl.BlockSpec((1, 32), lambda i: (0, 0))],
        out_specs=col_spec,
        out_shape=jax.ShapeDtypeStruct((B, 1), jnp.float32),
    )(u128, l128, p128, b128, usel, lsel, psel, blane, gt)


def kernel(user_ids, last_items, pre_items, user_table, item_table,
           global_transition, item_biases):
    B = user_ids.shape[0]
    V, D = user_table.shape
    uid = user_ids.astype(jnp.int32)
    lid = last_items.astype(jnp.int32)
    pid = pre_items.astype(jnp.int32)
    uidx2 = (uid >> 2).reshape(B // 128, 128)
    lidx2 = (lid >> 2).reshape(B // 128, 128)
    pidx2 = (pid >> 2).reshape(B // 128, 128)
    bidx2 = (pid >> 7).reshape(B // 128, 128)
    ut2 = user_table.reshape(V * D // 128, 128)
    it2 = item_table.reshape(V * D // 128, 128)
    nbias = item_biases.shape[0]
    pad = (-nbias) % 128
    bias2 = jnp.concatenate(
        [item_biases.reshape(-1), jnp.zeros((pad,), jnp.float32)]
    ).reshape((nbias + pad) // 128, 128)
    u128, l128, p128, b128 = _sc_gather(uidx2, lidx2, pidx2, bidx2,
                                        ut2, it2, bias2)
    out = _tc_math(u128, l128, p128, b128,
                   (uid & 3).reshape(B, 1), (lid & 3).reshape(B, 1),
                   (pid & 3).reshape(B, 1), (pid & 127).reshape(B, 1),
                   global_transition)
    return out.reshape(B)
